# Initial kernel scaffold; baseline (speedup 1.0000x reference)
import jax
import jax.numpy as jnp
from jax.experimental import pallas as pl


def kernel(x, edge_index, graph_indices, batch_ptr, params):
    raise NotImplementedError("write your pallas kernel here")


# speedup vs baseline: 11.8529x; 11.8529x over previous
import functools

import jax
import jax.numpy as jnp
from jax import lax
from jax.experimental import pallas as pl
from jax.experimental.pallas import tpu as pltpu
from jax.experimental.pallas import tpu_sc as plsc

N = 10000
E = 320000
G = 16
D = 128

NC = 2
NS = 16
NW = NC * NS
EPT = E // NW
CH = 125
NCHUNK = EPT // CH
HALF = NCHUNK // 2
ROW_STRIDE = 624
ROW_SPAN = 640

_mesh = plsc.VectorSubcoreMesh(core_axis_name="c", subcore_axis_name="s")


def _sc_agg_wide(y, src3, dst3, zeros):
    @functools.partial(
        pl.kernel,
        out_type=jax.ShapeDtypeStruct((NC, N, D), jnp.float32),
        mesh=_mesh,
        scratch_types=[
            pltpu.VMEM((HALF, CH), jnp.int32),
            pltpu.VMEM((HALF, CH), jnp.int32),
            pltpu.VMEM((2, CH, D), jnp.float32),
            pltpu.VMEM_SHARED((N, D), jnp.float32),
            pltpu.SemaphoreType.DMA,
            pltpu.SemaphoreType.DMA,
        ],
    )
    def k(y_hbm, src_hbm, dst_hbm, zero_hbm, out_hbm, src_v, dst_v, rows, acc,
          sem0, sem1):
        c = lax.axis_index("c")
        s = lax.axis_index("s")
        wid = s * NC + c
        sems = [sem0, sem1]

        base = pl.multiple_of(s * ROW_STRIDE, 8)
        pltpu.sync_copy(zero_hbm.at[pl.ds(base, ROW_SPAN)],
                        acc.at[pl.ds(base, ROW_SPAN)])
        plsc.subcore_barrier()

        for h in range(2):
            pltpu.sync_copy(src_hbm.at[wid, pl.ds(h * HALF, HALF)], src_v)
            pltpu.sync_copy(dst_hbm.at[wid, pl.ds(h * HALF, HALF)], dst_v)
            gathers = [
                pltpu.async_copy(y_hbm.at[src_v.at[0]], rows.at[0], sem0),
                pltpu.async_copy(y_hbm.at[src_v.at[1]], rows.at[1], sem1),
            ]
            for j in range(HALF):
                b = j % 2
                gathers[b].wait()
                pltpu.sync_copy(rows.at[b], acc.at[dst_v.at[j]], add=True)
                if j + 2 < HALF:
                    gathers[b] = pltpu.async_copy(
                        y_hbm.at[src_v.at[j + 2]], rows.at[b], sems[b])
        plsc.subcore_barrier()

        pltpu.sync_copy(acc.at[pl.ds(base, ROW_SPAN)],
                        out_hbm.at[c, pl.ds(base, ROW_SPAN)])

    return k(y, src3, dst3, zeros)


def _sc_counts(dst_t):
    @functools.partial(
        pl.kernel,
        out_type=jax.ShapeDtypeStruct((NW, N), jnp.float32),
        mesh=_mesh,
        compiler_params=pltpu.CompilerParams(needs_layout_passes=False),
        scratch_types=[
            pltpu.VMEM((EPT,), jnp.int32),
            pltpu.VMEM((N,), jnp.float32),
        ],
    )
    def k(dst_hbm, out_hbm, dst_v, acc):
        c = lax.axis_index("c")
        s = lax.axis_index("s")
        wid = s * NC + c
        pltpu.sync_copy(dst_hbm.at[wid], dst_v)
        zero16 = jnp.zeros((16,), jnp.float32)
        one16 = jnp.ones((16,), jnp.float32)

        def zbody(i, _):
            acc[pl.ds(i * 16, 16)] = zero16
            return 0

        lax.fori_loop(0, N // 16, zbody, 0)

        def body(e, _):
            idx = dst_v[pl.ds(e * 16, 16)]
            plsc.addupdate_scatter(acc, [idx], one16)
            return 0

        lax.fori_loop(0, EPT // 16, body, 0)
        pltpu.sync_copy(acc, out_hbm.at[wid])

    return k(dst_t)


def _sc_agg_scalar(a, src_t, dst_t):
    @functools.partial(
        pl.kernel,
        out_type=jax.ShapeDtypeStruct((NW, N), jnp.float32),
        mesh=_mesh,
        compiler_params=pltpu.CompilerParams(needs_layout_passes=False),
        scratch_types=[
            pltpu.VMEM((EPT,), jnp.int32),
            pltpu.VMEM((EPT,), jnp.int32),
            pltpu.VMEM((N,), jnp.float32),
            pltpu.VMEM((N,), jnp.float32),
        ],
    )
    def k(a_hbm, src_hbm, dst_hbm, out_hbm, src_v, dst_v, a_v, acc):
        c = lax.axis_index("c")
        s = lax.axis_index("s")
        wid = s * NC + c
        pltpu.sync_copy(src_hbm.at[wid], src_v)
        pltpu.sync_copy(dst_hbm.at[wid], dst_v)
        pltpu.sync_copy(a_hbm, a_v)
        zero16 = jnp.zeros((16,), jnp.float32)

        def zbody(i, _):
            acc[pl.ds(i * 16, 16)] = zero16
            return 0

        lax.fori_loop(0, N // 16, zbody, 0)

        def body(e, _):
            vals = plsc.load_gather(a_v, [src_v[pl.ds(e * 16, 16)]])
            plsc.addupdate_scatter(acc, [dst_v[pl.ds(e * 16, 16)]], vals)
            return 0

        lax.fori_loop(0, EPT // 16, body, 0)
        pltpu.sync_copy(acc, out_hbm.at[wid])

    return k(a, src_t, dst_t)


_R = 2000


def _tc_inv(cntp):
    def body(c_ref, o_ref):
        cnt = jnp.sum(c_ref[...], axis=0)
        o_ref[...] = (1.0 / jnp.maximum(cnt, 1.0))[:, None]

    return pl.pallas_call(
        body,
        out_shape=jax.ShapeDtypeStruct((N, 1), jnp.float32),
    )(cntp)


def _tc_dense(z, inv_deg, x, wlt, b, wrt, relu):
    def body(z0_ref, z1_ref, inv_ref, x_ref, wl_ref, b_ref, wr_ref, o_ref):
        mean = (z0_ref[...] + z1_ref[...]) * inv_ref[...]
        out = (jnp.dot(mean, wl_ref[...], preferred_element_type=jnp.float32)
               + b_ref[...]
               + jnp.dot(x_ref[...], wr_ref[...],
                         preferred_element_type=jnp.float32))
        if relu:
            out = jnp.maximum(out, 0.0)
        o_ref[...] = out

    grid = N // _R
    return pl.pallas_call(
        body,
        grid=(grid,),
        in_specs=[
            pl.BlockSpec((_R, D), lambda i: (i, 0)),
            pl.BlockSpec((_R, D), lambda i: (i, 0)),
            pl.BlockSpec((_R, 1), lambda i: (i, 0)),
            pl.BlockSpec((_R, D), lambda i: (i, 0)),
            pl.BlockSpec((D, D), lambda i: (0, 0)),
            pl.BlockSpec((1, D), lambda i: (0, 0)),
            pl.BlockSpec((D, D), lambda i: (0, 0)),
        ],
        out_specs=pl.BlockSpec((_R, D), lambda i: (i, 0)),
        out_shape=jax.ShapeDtypeStruct((N, D), jnp.float32),
    )(z[0], z[1], inv_deg, x, wlt, b, wrt)


def _tc_heads(z, inv_deg, emb, wplt, bp, wprt, wvlt, bv, wvrt, wp1lt):
    def body(z0_ref, z1_ref, inv_ref, x_ref, wpl_ref, bp_ref, wpr_ref,
             wvl_ref, bv_ref, wvr_ref, wp1_ref, hp_ref, hv_ref, a_ref):
        mean = (z0_ref[...] + z1_ref[...]) * inv_ref[...]
        x = x_ref[...]
        hp = jnp.maximum(
            jnp.dot(mean, wpl_ref[...], preferred_element_type=jnp.float32)
            + bp_ref[...]
            + jnp.dot(x, wpr_ref[...], preferred_element_type=jnp.float32),
            0.0)
        hv = jnp.maximum(
            jnp.dot(mean, wvl_ref[...], preferred_element_type=jnp.float32)
            + bv_ref[...]
            + jnp.dot(x, wvr_ref[...], preferred_element_type=jnp.float32),
            0.0)
        hp_ref[...] = hp
        hv_ref[...] = hv
        a_ref[...] = jnp.dot(hp, wp1_ref[...],
                             preferred_element_type=jnp.float32)

    grid = N // _R
    return pl.pallas_call(
        body,
        grid=(grid,),
        in_specs=[
            pl.BlockSpec((_R, D), lambda i: (i, 0)),
            pl.BlockSpec((_R, D), lambda i: (i, 0)),
            pl.BlockSpec((_R, 1), lambda i: (i, 0)),
            pl.BlockSpec((_R, D), lambda i: (i, 0)),
            pl.BlockSpec((D, D), lambda i: (0, 0)),
            pl.BlockSpec((1, D), lambda i: (0, 0)),
            pl.BlockSpec((D, D), lambda i: (0, 0)),
            pl.BlockSpec((D, D), lambda i: (0, 0)),
            pl.BlockSpec((1, D), lambda i: (0, 0)),
            pl.BlockSpec((D, D), lambda i: (0, 0)),
            pl.BlockSpec((D, 1), lambda i: (0, 0)),
        ],
        out_specs=[
            pl.BlockSpec((_R, D), lambda i: (i, 0)),
            pl.BlockSpec((_R, D), lambda i: (i, 0)),
            pl.BlockSpec((_R, 1), lambda i: (i, 0)),
        ],
        out_shape=[
            jax.ShapeDtypeStruct((N, D), jnp.float32),
            jax.ShapeDtypeStruct((N, D), jnp.float32),
            jax.ShapeDtypeStruct((N, 1), jnp.float32),
        ],
    )(z[0], z[1], inv_deg, emb, wplt, bp, wprt, wvlt, bv, wvrt, wp1lt)


def _tc_final(z, inv_deg, hv, hp, za, wv1lt, bv1, wv1rt, bp1, wp1rt,
              w1t, b1, w2t, b2):
    def body(z0_ref, z1_ref, inv_ref, hv_ref, hp_ref, za_ref, wvl_ref,
             bv_ref, wvr_ref, bp_ref, wpr_ref, w1_ref, b1_ref, w2_ref,
             b2_ref, pi_ref, val_ref):
        inv = inv_ref[...]
        mean = (z0_ref[...] + z1_ref[...]) * inv
        ve = (jnp.dot(mean, wvl_ref[...], preferred_element_type=jnp.float32)
              + bv_ref[...]
              + jnp.dot(hv_ref[...], wvr_ref[...],
                        preferred_element_type=jnp.float32))
        pi = (jnp.sum(za_ref[...], axis=0)[:, None] * inv
              + bp_ref[...]
              + jnp.dot(hp_ref[...], wpr_ref[...],
                        preferred_element_type=jnp.float32))
        pi_ref[...] = pi

        per = N // G
        sums, maxs, mins = [], [], []
        for g in range(G):
            blk = ve[g * per:(g + 1) * per, :]
            sums.append(jnp.sum(blk, axis=0, keepdims=True))
            maxs.append(jnp.max(blk, axis=0, keepdims=True))
            mins.append(jnp.min(blk, axis=0, keepdims=True))
        gs = jnp.concatenate(sums, axis=0)
        gmax = jnp.concatenate(maxs, axis=0)
        gmin = jnp.concatenate(mins, axis=0)
        gmean = gs / float(per)
        gp = jnp.concatenate([gs, gmax, gmin, gmean], axis=1)
        h = jnp.maximum(
            jnp.dot(gp, w1_ref[...], preferred_element_type=jnp.float32)
            + b1_ref[...], 0.0)
        val_ref[...] = jnp.tanh(
            jnp.dot(h, w2_ref[...], preferred_element_type=jnp.float32)
            + b2_ref[...])

    return pl.pallas_call(
        body,
        out_shape=[
            jax.ShapeDtypeStruct((N, 1), jnp.float32),
            jax.ShapeDtypeStruct((G, 1), jnp.float32),
        ],
    )(z[0], z[1], inv_deg, hv, hp, za, wv1lt, bv1, wv1rt, bp1, wp1rt,
      w1t, b1, w2t, b2)


def kernel(x, edge_index, graph_indices, batch_ptr, params):
    src = edge_index[0]
    dst = edge_index[1]
    src3 = src.reshape(NW, NCHUNK, CH)
    dst3 = dst.reshape(NW, NCHUNK, CH)
    src_t = src.reshape(NW, EPT)
    dst_t = dst.reshape(NW, EPT)
    zeros = jnp.zeros((N, D), jnp.float32)

    def wl(p):
        return p["lin_l"]["W"].T

    def bl(p):
        return p["lin_l"]["b"].reshape(1, -1)

    def wr(p):
        return p["lin_r"]["W"].T

    gnn = params["gnn"]
    ph = params["policy_head"]
    vh = params["value_head"]
    vm = params["value_mlp"]

    cntp = _sc_counts(dst_t)
    inv_deg = _tc_inv(cntp)

    h = x
    for i in range(3):
        z = _sc_agg_wide(h, src3, dst3, zeros)
        h = _tc_dense(z, inv_deg, h, wl(gnn[i]), bl(gnn[i]), wr(gnn[i]),
                      relu=(i != 2))

    z = _sc_agg_wide(h, src3, dst3, zeros)
    hp, hv, a = _tc_heads(z, inv_deg, h, wl(ph[0]), bl(ph[0]), wr(ph[0]),
                          wl(vh[0]), bl(vh[0]), wr(vh[0]), wl(ph[1]))

    z5 = _sc_agg_wide(hv, src3, dst3, zeros)
    za = _sc_agg_scalar(a.reshape(N), src_t, dst_t)

    pi, value = _tc_final(
        z5, inv_deg, hv, hp, za,
        wl(vh[1]), bl(vh[1]), wr(vh[1]),
        ph[1]["lin_l"]["b"].reshape(1, 1), wr(ph[1]),
        vm[0]["W"].T, vm[0]["b"].reshape(1, -1),
        vm[1]["W"].T, vm[1]["b"].reshape(1, -1),
    )
    return pi, value
